# baseline (speedup 1.0000x reference)
import functools

import jax
import jax.numpy as jnp
from jax import lax
from jax.experimental import pallas as pl
from jax.experimental.pallas import tpu as pltpu
from jax.experimental.pallas import tpu_sc as plsc

B = 100000
D = 128
NW = 32
CHUNK = 128
PW = 3128
N_FULL = PW // CHUNK
CHUNKS = tuple((j * CHUNK, CHUNK) for j in range(N_FULL)) + (
    (N_FULL * CHUNK, PW - N_FULL * CHUNK),)
LAST_BASE = B - PW
NBUF = 7
GDEPTH = 6


def _sc_gather(ids, table):
    mesh = plsc.VectorSubcoreMesh(core_axis_name="c", subcore_axis_name="s")

    @functools.partial(
        pl.kernel,
        mesh=mesh,
        out_type=jax.ShapeDtypeStruct((B, D), jnp.float32),
        scratch_types=(
            [pltpu.VMEM((PW,), jnp.int32),
             pltpu.VMEM((NBUF, CHUNK, D), jnp.float32)]
            + [pltpu.SemaphoreType.DMA] * (1 + 2 * NBUF)
        ),
    )
    def k(ids_hbm, table_hbm, out_hbm, idx_v, rows_v, isem, *sems):
        gsems = sems[:NBUF]
        ssems = sems[NBUF:]
        wid = lax.axis_index("s") * 2 + lax.axis_index("c")
        base = jnp.where(wid == NW - 1, LAST_BASE, wid * PW)
        pltpu.async_copy(ids_hbm.at[pl.ds(base, PW)], idx_v, isem).wait()
        n = len(CHUNKS)
        gath = [None] * n
        scat = [None] * n

        def fire_gather(j):
            b = j % NBUF
            off, sz = CHUNKS[j]
            if j >= NBUF:
                scat[j - NBUF].wait()
            gath[j] = pltpu.async_copy(
                table_hbm.at[idx_v.at[pl.ds(off, sz)]],
                rows_v.at[b, pl.ds(0, sz)], gsems[b])

        def fire_scatter(j):
            b = j % NBUF
            off, sz = CHUNKS[j]
            gath[j].wait()
            scat[j] = pltpu.async_copy(
                rows_v.at[b, pl.ds(0, sz)],
                out_hbm.at[pl.ds(base + off, sz)], ssems[b])

        for j in range(GDEPTH):
            fire_gather(j)
        for j in range(n):
            if j + GDEPTH < n:
                fire_gather(j + GDEPTH)
            fire_scatter(j)
        for j in range(n - NBUF, n):
            scat[j].wait()

    return k(ids, table)


def kernel(entity_ids, table):
    ids = jnp.squeeze(entity_ids).astype(jnp.int32)
    return _sc_gather(ids, table)


# speedup vs baseline: 1.0138x; 1.0138x over previous
import functools

import jax
import jax.numpy as jnp
from jax import lax
from jax.experimental import pallas as pl
from jax.experimental.pallas import tpu as pltpu
from jax.experimental.pallas import tpu_sc as plsc

B = 100000
D = 128
NW = 32
CHUNK = 128
PW = 3128
N_FULL = PW // CHUNK
CHUNKS = tuple((j * CHUNK, CHUNK) for j in range(N_FULL)) + (
    (N_FULL * CHUNK, PW - N_FULL * CHUNK),)
LAST_BASE = B - PW
NBUF = 6
GDEPTH = 5


def _sc_gather(ids, table):
    mesh = plsc.VectorSubcoreMesh(core_axis_name="c", subcore_axis_name="s")

    @functools.partial(
        pl.kernel,
        mesh=mesh,
        out_type=jax.ShapeDtypeStruct((B, D), jnp.float32),
        scratch_types=(
            [pltpu.VMEM((PW,), jnp.int32),
             pltpu.VMEM((NBUF, CHUNK, D), jnp.float32)]
            + [pltpu.SemaphoreType.DMA] * (1 + 2 * NBUF)
        ),
    )
    def k(ids_hbm, table_hbm, out_hbm, idx_v, rows_v, isem, *sems):
        gsems = sems[:NBUF]
        ssems = sems[NBUF:]
        wid = lax.axis_index("s") * 2 + lax.axis_index("c")
        base = jnp.where(wid == NW - 1, LAST_BASE, wid * PW)
        pltpu.async_copy(ids_hbm.at[pl.ds(base, PW)], idx_v, isem).wait()
        n = len(CHUNKS)
        gath = [None] * n
        scat = [None] * n

        def fire_gather(j):
            b = j % NBUF
            off, sz = CHUNKS[j]
            if j >= NBUF:
                scat[j - NBUF].wait()
            gath[j] = pltpu.async_copy(
                table_hbm.at[idx_v.at[pl.ds(off, sz)]],
                rows_v.at[b, pl.ds(0, sz)], gsems[b])

        def fire_scatter(j):
            b = j % NBUF
            off, sz = CHUNKS[j]
            gath[j].wait()
            scat[j] = pltpu.async_copy(
                rows_v.at[b, pl.ds(0, sz)],
                out_hbm.at[pl.ds(base + off, sz)], ssems[b])

        for j in range(GDEPTH):
            fire_gather(j)
        for j in range(n):
            if j + GDEPTH < n:
                fire_gather(j + GDEPTH)
            fire_scatter(j)
        for j in range(n - NBUF, n):
            scat[j].wait()

    return k(ids, table)


def kernel(entity_ids, table):
    ids = jnp.squeeze(entity_ids).astype(jnp.int32)
    return _sc_gather(ids, table)
